# baseline (speedup 1.0000x reference)
import jax
import jax.numpy as jnp
from jax.experimental import pallas as pl
from jax.experimental.pallas import tpu as pltpu

_B = 8
_N = 32768
_S = 1024


_BBLK = 4


def _fps_kernel(x0_ref, x1_ref, x2_ref, cinit_ref, pts_ref, dist_ref):
    dist_ref[...] = jnp.full((_BBLK, _N), 1e10, jnp.float32)
    lane = jax.lax.broadcasted_iota(jnp.int32, (_BBLK, _N), 1)

    def body(t, c):
        c0, c1, c2 = c
        pts_ref[0, t] = jnp.concatenate([c0, c1, c2], axis=1)
        x0 = x0_ref[0]
        x1 = x1_ref[0]
        x2 = x2_ref[0]
        d0 = x0 - c0
        d1 = x1 - c1
        d2 = x2 - c2
        d = d0 * d0 + d1 * d1 + d2 * d2
        dist = jnp.minimum(dist_ref[...], d)
        dist_ref[...] = dist
        m = jnp.max(dist, axis=1, keepdims=True)
        cand = jnp.where(dist == m, lane, _N)
        idx = jnp.min(cand, axis=1, keepdims=True)
        sel = lane == idx
        n0 = jnp.sum(jnp.where(sel, x0, 0.0), axis=1, keepdims=True)
        n1 = jnp.sum(jnp.where(sel, x1, 0.0), axis=1, keepdims=True)
        n2 = jnp.sum(jnp.where(sel, x2, 0.0), axis=1, keepdims=True)
        return (n0, n1, n2)

    c0 = cinit_ref[0, :, 0:1]
    c1 = cinit_ref[0, :, 1:2]
    c2 = cinit_ref[0, :, 2:3]
    jax.lax.fori_loop(0, _S, body, (c0, c1, c2))


def _mlp_kernel(p_ref, w1_ref, b1_ref, w2_ref, b2_ref, w3_ref, b3_ref, out_ref):
    p = p_ref[...]
    h = jnp.dot(p, w1_ref[...], preferred_element_type=jnp.float32)
    h = jnp.maximum(h + b1_ref[...], 0.0)
    h = jnp.dot(h, w2_ref[...], preferred_element_type=jnp.float32)
    h = jnp.maximum(h + b2_ref[...], 0.0)
    h = jnp.dot(h, w3_ref[...], preferred_element_type=jnp.float32)
    out_ref[...] = h + b3_ref[...]


def kernel(x, W1, b1, W2, b2, W3, b3):
    B, N, _ = x.shape
    init_idx = jax.random.randint(jax.random.key(1), (B,), 0, N, dtype=jnp.int32)
    cinit = x[jnp.arange(B), init_idx, :]

    ncores = B // _BBLK
    x0 = x[:, :, 0].reshape(ncores, _BBLK, N)
    x1 = x[:, :, 1].reshape(ncores, _BBLK, N)
    x2 = x[:, :, 2].reshape(ncores, _BBLK, N)
    cinit3 = cinit.reshape(ncores, _BBLK, 3)

    pts = pl.pallas_call(
        _fps_kernel,
        grid=(ncores,),
        in_specs=[
            pl.BlockSpec((1, _BBLK, _N), lambda i: (i, 0, 0)),
            pl.BlockSpec((1, _BBLK, _N), lambda i: (i, 0, 0)),
            pl.BlockSpec((1, _BBLK, _N), lambda i: (i, 0, 0)),
            pl.BlockSpec((1, _BBLK, 3), lambda i: (i, 0, 0)),
        ],
        out_specs=pl.BlockSpec((1, _S, _BBLK, 3), lambda i: (i, 0, 0, 0)),
        out_shape=jax.ShapeDtypeStruct((ncores, _S, _BBLK, 3), jnp.float32),
        scratch_shapes=[pltpu.VMEM((_BBLK, _N), jnp.float32)],
        compiler_params=pltpu.CompilerParams(
            dimension_semantics=("parallel",),
        ),
    )(x0, x1, x2, cinit3)

    sampled = jnp.transpose(pts, (0, 2, 1, 3)).reshape(B, _S, 3)

    feats = pl.pallas_call(
        _mlp_kernel,
        out_shape=jax.ShapeDtypeStruct((B * _S, 256), jnp.float32),
    )(
        sampled.reshape(B * _S, 3),
        W1,
        b1.reshape(1, 64),
        W2,
        b2.reshape(1, 128),
        W3,
        b3.reshape(1, 256),
    )

    return sampled, feats.reshape(B, _S, 256)


# speedup vs baseline: 2.9760x; 2.9760x over previous
import jax
import jax.numpy as jnp
from jax.experimental import pallas as pl
from jax.experimental.pallas import tpu as pltpu

_B = 8
_N = 32768
_S = 1024
_C = 1024
_NC = _N // _C


def _fps_kernel(x0_ref, x1_ref, x2_ref, cinit_ref, pts_ref, dist_ref):
    dist_ref[...] = jnp.full((_B, _N), 1e10, jnp.float32)
    lane_c = jax.lax.broadcasted_iota(jnp.int32, (_B, _C), 1)

    def body(t, c):
        c0, c1, c2 = c
        pts_ref[t] = jnp.concatenate([c0, c1, c2], axis=1)
        macc = jnp.full((_B, _C), -1.0, jnp.float32)
        kacc = jnp.zeros((_B, _C), jnp.int32)
        e0 = jnp.zeros((_B, _C), jnp.float32)
        e1 = jnp.zeros((_B, _C), jnp.float32)
        e2 = jnp.zeros((_B, _C), jnp.float32)
        for k in range(_NC):
            sl = pl.ds(k * _C, _C)
            x0c = x0_ref[:, sl]
            x1c = x1_ref[:, sl]
            x2c = x2_ref[:, sl]
            d0 = x0c - c0
            d1 = x1c - c1
            d2 = x2c - c2
            d = d0 * d0 + d1 * d1 + d2 * d2
            dc = jnp.minimum(dist_ref[:, sl], d)
            dist_ref[:, sl] = dc
            gt = dc > macc
            macc = jnp.where(gt, dc, macc)
            kacc = jnp.where(gt, k, kacc)
            e0 = jnp.where(gt, x0c, e0)
            e1 = jnp.where(gt, x1c, e1)
            e2 = jnp.where(gt, x2c, e2)
        m = jnp.max(macc, axis=1, keepdims=True)
        cand = jnp.where(macc == m, kacc * _C + lane_c, _N)
        idx = jnp.min(cand, axis=1, keepdims=True)
        selc = cand == idx
        n0 = jnp.sum(jnp.where(selc, e0, 0.0), axis=1, keepdims=True)
        n1 = jnp.sum(jnp.where(selc, e1, 0.0), axis=1, keepdims=True)
        n2 = jnp.sum(jnp.where(selc, e2, 0.0), axis=1, keepdims=True)
        return (n0, n1, n2)

    c0 = cinit_ref[:, 0:1]
    c1 = cinit_ref[:, 1:2]
    c2 = cinit_ref[:, 2:3]
    jax.lax.fori_loop(0, _S, body, (c0, c1, c2))


def _mlp_kernel(p_ref, w1_ref, b1_ref, w2_ref, b2_ref, w3_ref, b3_ref, out_ref):
    p = p_ref[...]
    h = jnp.dot(p, w1_ref[...], preferred_element_type=jnp.float32)
    h = jnp.maximum(h + b1_ref[...], 0.0)
    h = jnp.dot(h, w2_ref[...], preferred_element_type=jnp.float32)
    h = jnp.maximum(h + b2_ref[...], 0.0)
    h = jnp.dot(h, w3_ref[...], preferred_element_type=jnp.float32)
    out_ref[...] = h + b3_ref[...]


def kernel(x, W1, b1, W2, b2, W3, b3):
    B, N, _ = x.shape
    init_idx = jax.random.randint(jax.random.key(1), (B,), 0, N, dtype=jnp.int32)
    cinit = x[jnp.arange(B), init_idx, :]

    x0 = x[:, :, 0]
    x1 = x[:, :, 1]
    x2 = x[:, :, 2]

    pts = pl.pallas_call(
        _fps_kernel,
        out_shape=jax.ShapeDtypeStruct((_S, B, 3), jnp.float32),
        scratch_shapes=[pltpu.VMEM((_B, _N), jnp.float32)],
    )(x0, x1, x2, cinit)

    sampled = jnp.transpose(pts, (1, 0, 2))

    feats = pl.pallas_call(
        _mlp_kernel,
        out_shape=jax.ShapeDtypeStruct((B * _S, 256), jnp.float32),
    )(
        sampled.reshape(B * _S, 3),
        W1,
        b1.reshape(1, 64),
        W2,
        b2.reshape(1, 128),
        W3,
        b3.reshape(1, 256),
    )

    return sampled, feats.reshape(B, _S, 256)


# speedup vs baseline: 3.2899x; 1.1055x over previous
import jax
import jax.numpy as jnp
from jax.experimental import pallas as pl
from jax.experimental.pallas import tpu as pltpu

_B = 8
_N = 32768
_S = 1024
_C = 256
_NC = _N // _C


def _fused_kernel(
    x0_ref,
    x1_ref,
    x2_ref,
    cinit_ref,
    w1_ref,
    b1_ref,
    w2_ref,
    b2_ref,
    w3_ref,
    b3_ref,
    pts_ref,
    feats_ref,
    dist_ref,
    ptss_ref,
):
    dist_ref[...] = jnp.full((_B, _N), 1e10, jnp.float32)
    lane_c = jax.lax.broadcasted_iota(jnp.int32, (_B, _C), 1)

    def body(t, c):
        c0, c1, c2 = c
        ptss_ref[pl.ds(_B * t, _B), :] = jnp.concatenate([c0, c1, c2], axis=1)
        macc = jnp.full((_B, _C), -1.0, jnp.float32)
        kacc = jnp.zeros((_B, _C), jnp.int32)
        e0 = jnp.zeros((_B, _C), jnp.float32)
        e1 = jnp.zeros((_B, _C), jnp.float32)
        e2 = jnp.zeros((_B, _C), jnp.float32)
        for k in range(_NC):
            sl = pl.ds(k * _C, _C)
            x0c = x0_ref[:, sl]
            x1c = x1_ref[:, sl]
            x2c = x2_ref[:, sl]
            d0 = x0c - c0
            d1 = x1c - c1
            d2 = x2c - c2
            d = d0 * d0 + d1 * d1 + d2 * d2
            dc = jnp.minimum(dist_ref[:, sl], d)
            dist_ref[:, sl] = dc
            gt = dc > macc
            macc = jnp.where(gt, dc, macc)
            kacc = jnp.where(gt, k, kacc)
            e0 = jnp.where(gt, x0c, e0)
            e1 = jnp.where(gt, x1c, e1)
            e2 = jnp.where(gt, x2c, e2)
        m = jnp.max(macc, axis=1, keepdims=True)
        cand = jnp.where(macc == m, kacc * _C + lane_c, _N)
        idx = jnp.min(cand, axis=1, keepdims=True)
        selc = cand == idx
        n0 = jnp.sum(jnp.where(selc, e0, 0.0), axis=1, keepdims=True)
        n1 = jnp.sum(jnp.where(selc, e1, 0.0), axis=1, keepdims=True)
        n2 = jnp.sum(jnp.where(selc, e2, 0.0), axis=1, keepdims=True)
        return (n0, n1, n2)

    c0 = cinit_ref[:, 0:1]
    c1 = cinit_ref[:, 1:2]
    c2 = cinit_ref[:, 2:3]
    jax.lax.fori_loop(0, _S, body, (c0, c1, c2))

    p3 = ptss_ref[...].reshape(_S, _B, 3)
    for b in range(_B):
        pb = p3[:, b, :]
        pts_ref[b] = pb
        h = jnp.dot(pb, w1_ref[...], preferred_element_type=jnp.float32)
        h = jnp.maximum(h + b1_ref[...], 0.0)
        h = jnp.dot(h, w2_ref[...], preferred_element_type=jnp.float32)
        h = jnp.maximum(h + b2_ref[...], 0.0)
        h = jnp.dot(h, w3_ref[...], preferred_element_type=jnp.float32)
        feats_ref[b] = h + b3_ref[...]


def kernel(x, W1, b1, W2, b2, W3, b3):
    B, N, _ = x.shape
    init_idx = jax.random.randint(jax.random.key(1), (B,), 0, N, dtype=jnp.int32)
    cinit = x[jnp.arange(B), init_idx, :]

    x0 = x[:, :, 0]
    x1 = x[:, :, 1]
    x2 = x[:, :, 2]

    sampled, feats = pl.pallas_call(
        _fused_kernel,
        out_shape=(
            jax.ShapeDtypeStruct((B, _S, 3), jnp.float32),
            jax.ShapeDtypeStruct((B, _S, 256), jnp.float32),
        ),
        scratch_shapes=[
            pltpu.VMEM((_B, _N), jnp.float32),
            pltpu.VMEM((_S * _B, 3), jnp.float32),
        ],
    )(
        x0,
        x1,
        x2,
        cinit,
        W1,
        b1.reshape(1, 64),
        W2,
        b2.reshape(1, 128),
        W3,
        b3.reshape(1, 256),
    )

    return sampled, feats
